# Initial kernel scaffold; baseline (speedup 1.0000x reference)
import jax
import jax.numpy as jnp
from jax.experimental import pallas as pl


def kernel(x, Wq, Wk, Wv, Wo, bo, wk_comp, wv_comp, w_pe, Wg, bg):
    raise NotImplementedError("write your pallas kernel here")


# speedup vs baseline: 1.7286x; 1.7286x over previous
import functools

import jax
import jax.numpy as jnp
from jax import lax
from jax.experimental import pallas as pl
from jax.experimental.pallas import tpu as pltpu
from jax.experimental.pallas import tpu_sc as plsc

S = 2048
D = 1024
H = 16
HD = 64
CB = 16
NBLK = S // CB
SB = 8
TK = 2
WIN = 256
BQ = 256
NI = S // BQ
SCALE = 1.0 / 8.0
IMP_COEF = SCALE / (H * S)
NEG = -1e9


def _proj_body(x_ref, wq_ref, wk_ref, wv_ref, wg_ref, bg_ref, wck_ref,
               wcv_ref, pek_ref, pev_ref,
               q_ref, k_ref, v_ref, ck_ref, cv_ref, g_ref, imp_ref,
               ck_acc, qsum):
    i = pl.program_id(0)
    xb = x_ref[...]
    qb = jnp.dot(xb, wq_ref[...], preferred_element_type=jnp.float32)
    kb = jnp.dot(xb, wk_ref[...], preferred_element_type=jnp.float32)
    vb = jnp.dot(xb, wv_ref[...], preferred_element_type=jnp.float32)
    q_ref[...] = qb
    k_ref[...] = kb
    v_ref[...] = vb
    ckb = jnp.dot(wck_ref[...], kb, preferred_element_type=jnp.float32) + pek_ref[...]
    cvb = jnp.dot(wcv_ref[...], vb, preferred_element_type=jnp.float32) + pev_ref[...]
    ck_ref[...] = ckb
    cv_ref[...] = cvb
    g_ref[...] = jax.nn.sigmoid(
        jnp.dot(xb, wg_ref[...], preferred_element_type=jnp.float32) + bg_ref[...])
    nb = BQ // CB
    ck_acc[pl.ds(i * nb, nb), :] = ckb
    part = jnp.sum(qb, axis=0, keepdims=True)

    @pl.when(i == 0)
    def _():
        qsum[...] = part
        imp_ref[...] = jnp.zeros_like(imp_ref)

    @pl.when(i > 0)
    def _():
        qsum[...] += part

    @pl.when(i == NI - 1)
    def _():
        imp_ref[...] = lax.dot_general(
            qsum[...], ck_acc[...], (((1,), (1,)), ((), ())),
            preferred_element_type=jnp.float32) * IMP_COEF


def _projections(x2, Wq, Wk, Wv, Wgp, bgp, Wck, Wcv, pek, pev):
    full = lambda shape: pl.BlockSpec(shape, lambda i: (0, 0))
    return pl.pallas_call(
        _proj_body,
        grid=(NI,),
        in_specs=[
            pl.BlockSpec((BQ, D), lambda i: (i, 0)),
            full((D, D)), full((D, D)), full((D, D)),
            full((D, 8)), full((1, 8)),
            full((CB, BQ)), full((CB, BQ)),
            full((1, D)), full((1, D)),
        ],
        out_specs=[
            pl.BlockSpec((BQ, D), lambda i: (i, 0)),
            pl.BlockSpec((BQ, D), lambda i: (i, 0)),
            pl.BlockSpec((BQ, D), lambda i: (i, 0)),
            pl.BlockSpec((BQ // CB, D), lambda i: (i, 0)),
            pl.BlockSpec((BQ // CB, D), lambda i: (i, 0)),
            pl.BlockSpec((BQ, 8), lambda i: (i, 0)),
            pl.BlockSpec((1, NBLK), lambda i: (0, 0)),
        ],
        out_shape=[
            jax.ShapeDtypeStruct((S, D), jnp.float32),
            jax.ShapeDtypeStruct((S, D), jnp.float32),
            jax.ShapeDtypeStruct((S, D), jnp.float32),
            jax.ShapeDtypeStruct((NBLK, D), jnp.float32),
            jax.ShapeDtypeStruct((NBLK, D), jnp.float32),
            jax.ShapeDtypeStruct((S, 8), jnp.float32),
            jax.ShapeDtypeStruct((1, NBLK), jnp.float32),
        ],
        scratch_shapes=[
            pltpu.VMEM((NBLK, D), jnp.float32),
            pltpu.VMEM((1, D), jnp.float32),
        ],
    )(x2, Wq, Wk, Wv, Wgp, bgp, Wck, Wcv, pek, pev)


def _topk_gather(imp, k, v):
    mesh = plsc.VectorSubcoreMesh(core_axis_name="c", subcore_axis_name="s")

    @functools.partial(
        pl.kernel,
        out_type=[
            jax.ShapeDtypeStruct((CB,), jnp.int32),
            jax.ShapeDtypeStruct((TK * SB, D), jnp.float32),
            jax.ShapeDtypeStruct((TK * SB, D), jnp.float32),
        ],
        mesh=mesh,
        scratch_types=[
            pltpu.VMEM((NBLK,), jnp.float32),
            pltpu.VMEM((CB,), jnp.int32),
            pltpu.VMEM((TK * SB, D), jnp.float32),
            pltpu.SemaphoreType.DMA,
        ],
    )
    def sel_kernel(imp_hbm, k_hbm, v_hbm, selpos_hbm, sk_hbm, sv_hbm,
                   imp_v, selpos_v, rows_v, sem):
        cid = lax.axis_index("c")
        sid = lax.axis_index("s")
        wid = sid * 2 + cid

        @pl.when(wid < 2)
        def _():
            pltpu.sync_copy(imp_hbm, imp_v)
            lane = lax.iota(jnp.int32, 16)
            neg = jnp.full((16,), -3.4e38, jnp.float32)
            big = jnp.full((16,), 2 ** 30, jnp.int32)
            dnums = lax.GatherDimensionNumbers(
                offset_dims=(), collapsed_slice_dims=(0,), start_index_map=(0,))

            def lperm(u, idx):
                return lax.gather(u, idx[:, None], dnums, slice_sizes=(1,),
                                  mode=lax.GatherScatterMode.PROMISE_IN_BOUNDS)

            def allreduce(u, op):
                for s in (8, 4, 2, 1):
                    u = op(u, lperm(u, lane ^ s))
                return u

            vs = [imp_v[pl.ds(j * 16, 16)] for j in range(NBLK // 16)]
            gs = [lane + j * 16 for j in range(NBLK // 16)]

            def top1(vals):
                m = functools.reduce(jnp.maximum, vals)
                mall = allreduce(m, jnp.maximum)
                cand = functools.reduce(jnp.minimum, [
                    jnp.where(vv == mall, gg, big) for vv, gg in zip(vals, gs)])
                return allreduce(cand, jnp.minimum)

            i1 = top1(vs)
            i2 = top1([jnp.where(gg == i1, neg, vv) for vv, gg in zip(vs, gs)])
            sel = jnp.where(lane < SB, i1, i2) * CB + (lane & (SB - 1))
            selpos_v[...] = sel

            @pl.when(wid == 0)
            def _():
                pltpu.sync_copy(selpos_v, selpos_hbm)
                pltpu.async_copy(k_hbm.at[selpos_v], rows_v, sem).wait()
                pltpu.sync_copy(rows_v, sk_hbm)

            @pl.when(wid == 1)
            def _():
                pltpu.async_copy(v_hbm.at[selpos_v], rows_v, sem).wait()
                pltpu.sync_copy(rows_v, sv_hbm)

    return sel_kernel(imp, k, v)


def _attn_body(q_ref, kp_ref, kc_ref, vp_ref, vc_ref, ck_ref, cv_ref,
               sk_ref, sv_ref, selpos_ref, g_ref, wo_ref, bo_ref, out_ref):
    i = pl.program_id(0)
    h2 = pl.program_id(1)
    rowpos = i * BQ + lax.broadcasted_iota(jnp.int32, (BQ, 1), 0)

    def dot_t(a, b):
        return lax.dot_general(a, b, (((1,), (1,)), ((), ())),
                               preferred_element_type=jnp.float32)

    def dot_n(a, b):
        return lax.dot_general(a, b, (((1,), (0,)), ((), ())),
                               preferred_element_type=jnp.float32)

    blk_end = (lax.broadcasted_iota(jnp.int32, (1, NBLK), 1) + 1) * CB
    mask1 = blk_end <= rowpos
    mask2 = selpos_ref[...] <= rowpos
    colid = lax.broadcasted_iota(jnp.int32, (1, BQ), 1)
    pa = jnp.maximum(i - 1, 0) * BQ + colid
    pb = i * BQ + colid
    ma = (pa <= rowpos) & (pa > rowpos - WIN) & (i > 0)
    mb = pb <= rowpos
    gb = g_ref[...]
    g0, g1, g2 = gb[:, 0:1], gb[:, 1:2], gb[:, 2:3]

    parts = []
    for t in range(2):
        sl = pl.ds(t * HD, HD)
        qb = q_ref[:, sl]

        s1 = dot_t(qb, ck_ref[:, sl]) * SCALE
        s1 = jnp.where(mask1, s1, NEG)
        m1 = jnp.max(s1, axis=1, keepdims=True)
        p1 = jnp.exp(s1 - m1)
        o1 = dot_n(p1, cv_ref[:, sl]) / jnp.sum(p1, axis=1, keepdims=True)

        s2 = dot_t(qb, sk_ref[:, sl]) * SCALE
        s2 = jnp.where(mask2, s2, NEG)
        m2 = jnp.max(s2, axis=1, keepdims=True)
        p2 = jnp.exp(s2 - m2)
        o2 = dot_n(p2, sv_ref[:, sl]) / jnp.sum(p2, axis=1, keepdims=True)

        s3a = dot_t(qb, kp_ref[:, sl]) * SCALE
        s3b = dot_t(qb, kc_ref[:, sl]) * SCALE
        s3a = jnp.where(ma, s3a, NEG)
        s3b = jnp.where(mb, s3b, NEG)
        m3 = jnp.maximum(jnp.max(s3a, axis=1, keepdims=True),
                         jnp.max(s3b, axis=1, keepdims=True))
        p3a = jnp.exp(s3a - m3)
        p3b = jnp.exp(s3b - m3)
        d3 = jnp.sum(p3a, axis=1, keepdims=True) + jnp.sum(p3b, axis=1, keepdims=True)
        o3 = (dot_n(p3a, vp_ref[:, sl]) + dot_n(p3b, vc_ref[:, sl])) / d3

        comb = g0 * o1 + g1 * o2 + g2 * o3
        parts.append(dot_n(comb, wo_ref[sl, :]))

    part = parts[0] + parts[1]

    @pl.when(h2 == 0)
    def _():
        out_ref[...] = part + bo_ref[...]

    @pl.when(h2 > 0)
    def _():
        out_ref[...] += part


def _attention(q, k, v, ck, cv, sk, sv, selpos, g, Wo, bo2):
    HP = 2 * HD
    return pl.pallas_call(
        _attn_body,
        grid=(NI, H // 2),
        in_specs=[
            pl.BlockSpec((BQ, HP), lambda i, h: (i, h)),
            pl.BlockSpec((BQ, HP), lambda i, h: (jnp.maximum(i - 1, 0), h)),
            pl.BlockSpec((BQ, HP), lambda i, h: (i, h)),
            pl.BlockSpec((BQ, HP), lambda i, h: (jnp.maximum(i - 1, 0), h)),
            pl.BlockSpec((BQ, HP), lambda i, h: (i, h)),
            pl.BlockSpec((NBLK, HP), lambda i, h: (0, h)),
            pl.BlockSpec((NBLK, HP), lambda i, h: (0, h)),
            pl.BlockSpec((TK * SB, HP), lambda i, h: (0, h)),
            pl.BlockSpec((TK * SB, HP), lambda i, h: (0, h)),
            pl.BlockSpec((1, TK * SB), lambda i, h: (0, 0)),
            pl.BlockSpec((BQ, 8), lambda i, h: (i, 0)),
            pl.BlockSpec((HP, D), lambda i, h: (h, 0)),
            pl.BlockSpec((1, D), lambda i, h: (0, 0)),
        ],
        out_specs=pl.BlockSpec((BQ, D), lambda i, h: (i, 0)),
        out_shape=jax.ShapeDtypeStruct((S, D), jnp.float32),
    )(q, k, k, v, v, ck, cv, sk, sv, selpos, g, Wo, bo2)


def kernel(x, Wq, Wk, Wv, Wo, bo, wk_comp, wv_comp, w_pe, Wg, bg):
    x2 = x[0]
    wkc = wk_comp[:, 0]
    wvc = wv_comp[:, 0]
    eye = jnp.eye(CB, dtype=jnp.float32)
    Wck = jnp.kron(eye, wkc[None, :])
    Wcv = jnp.kron(eye, wvc[None, :])
    pek = (wkc @ w_pe)[None, :]
    pev = (wvc @ w_pe)[None, :]
    Wgp = jnp.pad(Wg, ((0, 0), (0, 5)))
    bgp = jnp.pad(bg, (0, 5))[None, :]

    q, k, v, ck, cv, g, imp = _projections(
        x2, Wq, Wk, Wv, Wgp, bgp, Wck, Wcv, pek, pev)
    selpos, sk, sv = _topk_gather(imp.reshape(NBLK), k, v)
    out = _attention(q, k, v, ck, cv, sk, sv,
                     selpos.reshape(1, TK * SB), g, Wo, bo[None, :])
    return out[None]


# speedup vs baseline: 2.2270x; 1.2883x over previous
import functools

import jax
import jax.numpy as jnp
from jax import lax
from jax.experimental import pallas as pl
from jax.experimental.pallas import tpu as pltpu
from jax.experimental.pallas import tpu_sc as plsc

S = 2048
D = 1024
H = 16
HD = 64
CB = 16
NBLK = S // CB
SB = 8
TK = 2
WIN = 256
BQ = 256
NI = S // BQ
SCALE = 1.0 / 8.0
IMP_COEF = SCALE / (H * S)
NEG = -1e9


def _proj_body(x_ref, wq_ref, wk_ref, wv_ref, wg_ref, bg_ref, wck_ref,
               wcv_ref, pek_ref, pev_ref,
               q_ref, k_ref, v_ref, ck_ref, cv_ref, g_ref, imp_ref,
               ck_acc, qsum):
    i = pl.program_id(0)
    xb = x_ref[...]
    qb = jnp.dot(xb, wq_ref[...], preferred_element_type=jnp.float32)
    kb = jnp.dot(xb, wk_ref[...], preferred_element_type=jnp.float32)
    vb = jnp.dot(xb, wv_ref[...], preferred_element_type=jnp.float32)
    q_ref[...] = qb
    k_ref[...] = kb
    v_ref[...] = vb
    ckb = jnp.dot(wck_ref[...], kb, preferred_element_type=jnp.float32) + pek_ref[...]
    cvb = jnp.dot(wcv_ref[...], vb, preferred_element_type=jnp.float32) + pev_ref[...]
    ck_ref[...] = ckb
    cv_ref[...] = cvb
    g_ref[...] = jax.nn.sigmoid(
        jnp.dot(xb, wg_ref[...], preferred_element_type=jnp.float32) + bg_ref[...])
    nb = BQ // CB
    ck_acc[pl.ds(i * nb, nb), :] = ckb
    part = jnp.sum(qb, axis=0, keepdims=True)

    @pl.when(i == 0)
    def _():
        qsum[...] = part
        imp_ref[...] = jnp.zeros_like(imp_ref)

    @pl.when(i > 0)
    def _():
        qsum[...] += part

    @pl.when(i == NI - 1)
    def _():
        imp_ref[...] = lax.dot_general(
            qsum[...], ck_acc[...], (((1,), (1,)), ((), ())),
            preferred_element_type=jnp.float32) * IMP_COEF


def _projections(x2, Wq, Wk, Wv, Wgp, bgp, Wck, Wcv, pek, pev):
    full = lambda shape: pl.BlockSpec(shape, lambda i: (0, 0))
    return pl.pallas_call(
        _proj_body,
        grid=(NI,),
        in_specs=[
            pl.BlockSpec((BQ, D), lambda i: (i, 0)),
            full((D, D)), full((D, D)), full((D, D)),
            full((D, 8)), full((1, 8)),
            full((CB, BQ)), full((CB, BQ)),
            full((1, D)), full((1, D)),
        ],
        out_specs=[
            pl.BlockSpec((BQ, D), lambda i: (i, 0)),
            pl.BlockSpec((BQ, D), lambda i: (i, 0)),
            pl.BlockSpec((BQ, D), lambda i: (i, 0)),
            pl.BlockSpec((BQ // CB, D), lambda i: (i, 0)),
            pl.BlockSpec((BQ // CB, D), lambda i: (i, 0)),
            pl.BlockSpec((BQ, 8), lambda i: (i, 0)),
            pl.BlockSpec((1, NBLK), lambda i: (0, 0)),
        ],
        out_shape=[
            jax.ShapeDtypeStruct((S, D), jnp.float32),
            jax.ShapeDtypeStruct((S, D), jnp.float32),
            jax.ShapeDtypeStruct((S, D), jnp.float32),
            jax.ShapeDtypeStruct((NBLK, D), jnp.float32),
            jax.ShapeDtypeStruct((NBLK, D), jnp.float32),
            jax.ShapeDtypeStruct((S, 8), jnp.float32),
            jax.ShapeDtypeStruct((1, NBLK), jnp.float32),
        ],
        scratch_shapes=[
            pltpu.VMEM((NBLK, D), jnp.float32),
            pltpu.VMEM((1, D), jnp.float32),
        ],
    )(x2, Wq, Wk, Wv, Wgp, bgp, Wck, Wcv, pek, pev)


def _topk_gather(imp, k, v):
    mesh = plsc.VectorSubcoreMesh(core_axis_name="c", subcore_axis_name="s")

    @functools.partial(
        pl.kernel,
        out_type=[
            jax.ShapeDtypeStruct((CB,), jnp.int32),
            jax.ShapeDtypeStruct((TK * SB, D), jnp.float32),
            jax.ShapeDtypeStruct((TK * SB, D), jnp.float32),
        ],
        mesh=mesh,
        scratch_types=[
            pltpu.VMEM((NBLK,), jnp.float32),
            pltpu.VMEM((CB,), jnp.int32),
            pltpu.VMEM((TK * SB, D), jnp.float32),
            pltpu.SemaphoreType.DMA,
        ],
    )
    def sel_kernel(imp_hbm, k_hbm, v_hbm, selpos_hbm, sk_hbm, sv_hbm,
                   imp_v, selpos_v, rows_v, sem):
        cid = lax.axis_index("c")
        sid = lax.axis_index("s")
        wid = sid * 2 + cid

        @pl.when(wid < 2)
        def _():
            pltpu.sync_copy(imp_hbm, imp_v)
            lane = lax.iota(jnp.int32, 16)
            neg = jnp.full((16,), -3.4e38, jnp.float32)
            big = jnp.full((16,), 2 ** 30, jnp.int32)
            dnums = lax.GatherDimensionNumbers(
                offset_dims=(), collapsed_slice_dims=(0,), start_index_map=(0,))

            def lperm(u, idx):
                return lax.gather(u, idx[:, None], dnums, slice_sizes=(1,),
                                  mode=lax.GatherScatterMode.PROMISE_IN_BOUNDS)

            def allreduce(u, op):
                for s in (8, 4, 2, 1):
                    u = op(u, lperm(u, lane ^ s))
                return u

            vs = [imp_v[pl.ds(j * 16, 16)] for j in range(NBLK // 16)]
            gs = [lane + j * 16 for j in range(NBLK // 16)]

            def top1(vals):
                m = functools.reduce(jnp.maximum, vals)
                mall = allreduce(m, jnp.maximum)
                cand = functools.reduce(jnp.minimum, [
                    jnp.where(vv == mall, gg, big) for vv, gg in zip(vals, gs)])
                return allreduce(cand, jnp.minimum)

            i1 = top1(vs)
            i2 = top1([jnp.where(gg == i1, neg, vv) for vv, gg in zip(vs, gs)])
            sel = jnp.where(lane < SB, i1, i2) * CB + (lane & (SB - 1))
            selpos_v[...] = sel

            @pl.when(wid == 0)
            def _():
                pltpu.sync_copy(selpos_v, selpos_hbm)
                pltpu.async_copy(k_hbm.at[selpos_v], rows_v, sem).wait()
                pltpu.sync_copy(rows_v, sk_hbm)

            @pl.when(wid == 1)
            def _():
                pltpu.async_copy(v_hbm.at[selpos_v], rows_v, sem).wait()
                pltpu.sync_copy(rows_v, sv_hbm)

    return sel_kernel(imp, k, v)


def _attn_body(q_ref, kp_ref, kc_ref, vp_ref, vc_ref, ck_ref, cv_ref,
               sk_ref, sv_ref, selpos_ref, g_ref, wo_ref, bo_ref, out_ref):
    i = pl.program_id(0)
    rowpos = i * BQ + lax.broadcasted_iota(jnp.int32, (BQ, 1), 0)

    def dot_t(a, b):
        return lax.dot_general(a, b, (((1,), (1,)), ((), ())),
                               preferred_element_type=jnp.float32)

    def dot_n(a, b):
        return lax.dot_general(a, b, (((1,), (0,)), ((), ())),
                               preferred_element_type=jnp.float32)

    blk_end = (lax.broadcasted_iota(jnp.int32, (1, NBLK), 1) + 1) * CB
    mask1 = blk_end <= rowpos
    mask2 = selpos_ref[...] <= rowpos
    colid = lax.broadcasted_iota(jnp.int32, (1, BQ), 1)
    pa = jnp.maximum(i - 1, 0) * BQ + colid
    pb = i * BQ + colid
    ma = (pa <= rowpos) & (pa > rowpos - WIN) & (i > 0)
    mb = pb <= rowpos
    gb = g_ref[...]
    g0, g1, g2 = gb[:, 0:1], gb[:, 1:2], gb[:, 2:3]

    parts = []
    for t in range(H):
        sl = pl.ds(t * HD, HD)
        qb = q_ref[:, sl]

        s1 = dot_t(qb, ck_ref[:, sl]) * SCALE
        s1 = jnp.where(mask1, s1, NEG)
        m1 = jnp.max(s1, axis=1, keepdims=True)
        p1 = jnp.exp(s1 - m1)
        o1 = dot_n(p1, cv_ref[:, sl]) / jnp.sum(p1, axis=1, keepdims=True)

        s2 = dot_t(qb, sk_ref[:, sl]) * SCALE
        s2 = jnp.where(mask2, s2, NEG)
        m2 = jnp.max(s2, axis=1, keepdims=True)
        p2 = jnp.exp(s2 - m2)
        o2 = dot_n(p2, sv_ref[:, sl]) / jnp.sum(p2, axis=1, keepdims=True)

        s3a = dot_t(qb, kp_ref[:, sl]) * SCALE
        s3b = dot_t(qb, kc_ref[:, sl]) * SCALE
        s3a = jnp.where(ma, s3a, NEG)
        s3b = jnp.where(mb, s3b, NEG)
        m3 = jnp.maximum(jnp.max(s3a, axis=1, keepdims=True),
                         jnp.max(s3b, axis=1, keepdims=True))
        p3a = jnp.exp(s3a - m3)
        p3b = jnp.exp(s3b - m3)
        d3 = jnp.sum(p3a, axis=1, keepdims=True) + jnp.sum(p3b, axis=1, keepdims=True)
        o3 = (dot_n(p3a, vp_ref[:, sl]) + dot_n(p3b, vc_ref[:, sl])) / d3

        parts.append(g0 * o1 + g1 * o2 + g2 * o3)

    comb = jnp.concatenate(parts, axis=1)
    out_ref[...] = dot_n(comb, wo_ref[...]) + bo_ref[...]


def _attention(q, k, v, ck, cv, sk, sv, selpos, g, Wo, bo2):
    full = lambda shape: pl.BlockSpec(shape, lambda i: (0, 0))
    return pl.pallas_call(
        _attn_body,
        grid=(NI,),
        in_specs=[
            pl.BlockSpec((BQ, D), lambda i: (i, 0)),
            pl.BlockSpec((BQ, D), lambda i: (jnp.maximum(i - 1, 0), 0)),
            pl.BlockSpec((BQ, D), lambda i: (i, 0)),
            pl.BlockSpec((BQ, D), lambda i: (jnp.maximum(i - 1, 0), 0)),
            pl.BlockSpec((BQ, D), lambda i: (i, 0)),
            full((NBLK, D)),
            full((NBLK, D)),
            full((TK * SB, D)),
            full((TK * SB, D)),
            full((1, TK * SB)),
            pl.BlockSpec((BQ, 8), lambda i: (i, 0)),
            full((D, D)),
            full((1, D)),
        ],
        out_specs=pl.BlockSpec((BQ, D), lambda i: (i, 0)),
        out_shape=jax.ShapeDtypeStruct((S, D), jnp.float32),
    )(q, k, k, v, v, ck, cv, sk, sv, selpos, g, Wo, bo2)


def kernel(x, Wq, Wk, Wv, Wo, bo, wk_comp, wv_comp, w_pe, Wg, bg):
    x2 = x[0]
    wkc = wk_comp[:, 0]
    wvc = wv_comp[:, 0]
    eye = jnp.eye(CB, dtype=jnp.float32)
    Wck = jnp.kron(eye, wkc[None, :])
    Wcv = jnp.kron(eye, wvc[None, :])
    pek = (wkc @ w_pe)[None, :]
    pev = (wvc @ w_pe)[None, :]
    Wgp = jnp.pad(Wg, ((0, 0), (0, 5)))
    bgp = jnp.pad(bg, (0, 5))[None, :]

    q, k, v, ck, cv, g, imp = _projections(
        x2, Wq, Wk, Wv, Wgp, bgp, Wck, Wcv, pek, pev)
    selpos, sk, sv = _topk_gather(imp.reshape(NBLK), k, v)
    out = _attention(q, k, v, ck, cv, sk, sv,
                     selpos.reshape(1, TK * SB), g, Wo, bo[None, :])
    return out[None]


# speedup vs baseline: 2.2689x; 1.0188x over previous
import functools

import jax
import jax.numpy as jnp
from jax import lax
from jax.experimental import pallas as pl
from jax.experimental.pallas import tpu as pltpu
from jax.experimental.pallas import tpu_sc as plsc

S = 2048
D = 1024
H = 16
HD = 64
CB = 16
NBLK = S // CB
SB = 8
TK = 2
WIN = 256
BQ = 256
NI = S // BQ
SCALE = 1.0 / 8.0
IMP_COEF = SCALE / (H * S)
NEG = -1e9


def _proj_body(x_ref, wq_ref, wk_ref, wv_ref, wg_ref, bg_ref, wkc_ref,
               wvc_ref, wpe_ref,
               q_ref, k_ref, v_ref, ck_ref, cv_ref, g_ref, imp_ref,
               ck_acc, qsum):
    i = pl.program_id(0)
    xb = x_ref[...]
    qb = jnp.dot(xb, wq_ref[...], preferred_element_type=jnp.float32)
    kb = jnp.dot(xb, wk_ref[...], preferred_element_type=jnp.float32)
    vb = jnp.dot(xb, wv_ref[...], preferred_element_type=jnp.float32)
    q_ref[...] = qb
    k_ref[...] = kb
    v_ref[...] = vb
    row16 = lax.broadcasted_iota(jnp.int32, (CB, BQ), 0)
    col16 = lax.broadcasted_iota(jnp.int32, (CB, BQ), 1)
    onblk = (col16 >> 4) == row16
    wkrow = lax.transpose(wkc_ref[...], (1, 0))
    wvrow = lax.transpose(wvc_ref[...], (1, 0))
    wktile = jnp.concatenate([wkrow] * (BQ // CB), axis=1)
    wvtile = jnp.concatenate([wvrow] * (BQ // CB), axis=1)
    wck = jnp.where(onblk, wktile, 0.0)
    wcv = jnp.where(onblk, wvtile, 0.0)
    pek = jnp.dot(wkrow, wpe_ref[...], preferred_element_type=jnp.float32)
    pev = jnp.dot(wvrow, wpe_ref[...], preferred_element_type=jnp.float32)
    ckb = jnp.dot(wck, kb, preferred_element_type=jnp.float32) + pek
    cvb = jnp.dot(wcv, vb, preferred_element_type=jnp.float32) + pev
    ck_ref[...] = ckb
    cv_ref[...] = cvb
    g_ref[...] = jax.nn.sigmoid(
        jnp.dot(xb, wg_ref[...], preferred_element_type=jnp.float32) + bg_ref[...])
    nb = BQ // CB
    ck_acc[pl.ds(i * nb, nb), :] = ckb
    part = jnp.sum(qb, axis=0, keepdims=True)

    @pl.when(i == 0)
    def _():
        qsum[...] = part
        imp_ref[...] = jnp.zeros_like(imp_ref)

    @pl.when(i > 0)
    def _():
        qsum[...] += part

    @pl.when(i == NI - 1)
    def _():
        imp_ref[...] = lax.dot_general(
            qsum[...], ck_acc[...], (((1,), (1,)), ((), ())),
            preferred_element_type=jnp.float32) * IMP_COEF


def _projections(x2, Wq, Wk, Wv, Wg, bg2, wk_comp, wv_comp, w_pe):
    full = lambda shape: pl.BlockSpec(shape, lambda i: (0, 0))
    return pl.pallas_call(
        _proj_body,
        grid=(NI,),
        in_specs=[
            pl.BlockSpec((BQ, D), lambda i: (i, 0)),
            full((D, D)), full((D, D)), full((D, D)),
            full((D, 3)), full((1, 3)),
            full((CB, 1)), full((CB, 1)),
            full((CB, D)),
        ],
        out_specs=[
            pl.BlockSpec((BQ, D), lambda i: (i, 0)),
            pl.BlockSpec((BQ, D), lambda i: (i, 0)),
            pl.BlockSpec((BQ, D), lambda i: (i, 0)),
            pl.BlockSpec((BQ // CB, D), lambda i: (i, 0)),
            pl.BlockSpec((BQ // CB, D), lambda i: (i, 0)),
            pl.BlockSpec((BQ, 3), lambda i: (i, 0)),
            pl.BlockSpec((1, NBLK), lambda i: (0, 0)),
        ],
        out_shape=[
            jax.ShapeDtypeStruct((S, D), jnp.float32),
            jax.ShapeDtypeStruct((S, D), jnp.float32),
            jax.ShapeDtypeStruct((S, D), jnp.float32),
            jax.ShapeDtypeStruct((NBLK, D), jnp.float32),
            jax.ShapeDtypeStruct((NBLK, D), jnp.float32),
            jax.ShapeDtypeStruct((S, 3), jnp.float32),
            jax.ShapeDtypeStruct((1, NBLK), jnp.float32),
        ],
        scratch_shapes=[
            pltpu.VMEM((NBLK, D), jnp.float32),
            pltpu.VMEM((1, D), jnp.float32),
        ],
    )(x2, Wq, Wk, Wv, Wg, bg2, wk_comp, wv_comp, w_pe)


def _topk_gather(imp, k, v):
    mesh = plsc.VectorSubcoreMesh(core_axis_name="c", subcore_axis_name="s")

    @functools.partial(
        pl.kernel,
        out_type=[
            jax.ShapeDtypeStruct((CB,), jnp.int32),
            jax.ShapeDtypeStruct((TK * SB, D), jnp.float32),
            jax.ShapeDtypeStruct((TK * SB, D), jnp.float32),
        ],
        mesh=mesh,
        scratch_types=[
            pltpu.VMEM((NBLK,), jnp.float32),
            pltpu.VMEM((CB,), jnp.int32),
            pltpu.VMEM((TK * SB, D), jnp.float32),
            pltpu.SemaphoreType.DMA,
        ],
    )
    def sel_kernel(imp_hbm, k_hbm, v_hbm, selpos_hbm, sk_hbm, sv_hbm,
                   imp_v, selpos_v, rows_v, sem):
        cid = lax.axis_index("c")
        sid = lax.axis_index("s")
        wid = sid * 2 + cid

        @pl.when(wid < 2)
        def _():
            pltpu.sync_copy(imp_hbm, imp_v)
            lane = lax.iota(jnp.int32, 16)
            neg = jnp.full((16,), -3.4e38, jnp.float32)
            big = jnp.full((16,), 2 ** 30, jnp.int32)
            dnums = lax.GatherDimensionNumbers(
                offset_dims=(), collapsed_slice_dims=(0,), start_index_map=(0,))

            def lperm(u, idx):
                return lax.gather(u, idx[:, None], dnums, slice_sizes=(1,),
                                  mode=lax.GatherScatterMode.PROMISE_IN_BOUNDS)

            def allreduce(u, op):
                for s in (8, 4, 2, 1):
                    u = op(u, lperm(u, lane ^ s))
                return u

            vs = [imp_v[pl.ds(j * 16, 16)] for j in range(NBLK // 16)]
            gs = [lane + j * 16 for j in range(NBLK // 16)]

            def top1(vals):
                m = functools.reduce(jnp.maximum, vals)
                mall = allreduce(m, jnp.maximum)
                cand = functools.reduce(jnp.minimum, [
                    jnp.where(vv == mall, gg, big) for vv, gg in zip(vals, gs)])
                return allreduce(cand, jnp.minimum)

            i1 = top1(vs)
            i2 = top1([jnp.where(gg == i1, neg, vv) for vv, gg in zip(vs, gs)])
            sel = jnp.where(lane < SB, i1, i2) * CB + (lane & (SB - 1))
            selpos_v[...] = sel

            @pl.when(wid == 0)
            def _():
                pltpu.sync_copy(selpos_v, selpos_hbm)
                pltpu.async_copy(k_hbm.at[selpos_v], rows_v, sem).wait()
                pltpu.sync_copy(rows_v, sk_hbm)

            @pl.when(wid == 1)
            def _():
                pltpu.async_copy(v_hbm.at[selpos_v], rows_v, sem).wait()
                pltpu.sync_copy(rows_v, sv_hbm)

    return sel_kernel(imp, k, v)


def _attn_body(q_ref, kp_ref, kc_ref, vp_ref, vc_ref, ck_ref, cv_ref,
               sk_ref, sv_ref, selpos_ref, g_ref, wo_ref, bo_ref, out_ref):
    i = pl.program_id(0)
    rowpos = i * BQ + lax.broadcasted_iota(jnp.int32, (BQ, 1), 0)

    def dot_t(a, b):
        return lax.dot_general(a, b, (((1,), (1,)), ((), ())),
                               preferred_element_type=jnp.float32)

    def dot_n(a, b):
        return lax.dot_general(a, b, (((1,), (0,)), ((), ())),
                               preferred_element_type=jnp.float32)

    blk_end = (lax.broadcasted_iota(jnp.int32, (1, NBLK), 1) + 1) * CB
    mask1 = blk_end <= rowpos
    mask2 = selpos_ref[...] <= rowpos
    colid = lax.broadcasted_iota(jnp.int32, (1, BQ), 1)
    pa = jnp.maximum(i - 1, 0) * BQ + colid
    pb = i * BQ + colid
    ma = (pa <= rowpos) & (pa > rowpos - WIN) & (i > 0)
    mb = pb <= rowpos
    gb = g_ref[...]
    g0, g1, g2 = gb[:, 0:1], gb[:, 1:2], gb[:, 2:3]

    parts = []
    for t in range(H):
        sl = pl.ds(t * HD, HD)
        qb = q_ref[:, sl]

        s1 = dot_t(qb, ck_ref[:, sl]) * SCALE
        s1 = jnp.where(mask1, s1, NEG)
        m1 = jnp.max(s1, axis=1, keepdims=True)
        p1 = jnp.exp(s1 - m1)
        o1 = dot_n(p1, cv_ref[:, sl]) / jnp.sum(p1, axis=1, keepdims=True)

        s2 = dot_t(qb, sk_ref[:, sl]) * SCALE
        s2 = jnp.where(mask2, s2, NEG)
        m2 = jnp.max(s2, axis=1, keepdims=True)
        p2 = jnp.exp(s2 - m2)
        o2 = dot_n(p2, sv_ref[:, sl]) / jnp.sum(p2, axis=1, keepdims=True)

        s3a = dot_t(qb, kp_ref[:, sl]) * SCALE
        s3b = dot_t(qb, kc_ref[:, sl]) * SCALE
        s3a = jnp.where(ma, s3a, NEG)
        s3b = jnp.where(mb, s3b, NEG)
        m3 = jnp.maximum(jnp.max(s3a, axis=1, keepdims=True),
                         jnp.max(s3b, axis=1, keepdims=True))
        p3a = jnp.exp(s3a - m3)
        p3b = jnp.exp(s3b - m3)
        d3 = jnp.sum(p3a, axis=1, keepdims=True) + jnp.sum(p3b, axis=1, keepdims=True)
        o3 = (dot_n(p3a, vp_ref[:, sl]) + dot_n(p3b, vc_ref[:, sl])) / d3

        parts.append(g0 * o1 + g1 * o2 + g2 * o3)

    comb = jnp.concatenate(parts, axis=1)
    out_ref[...] = dot_n(comb, wo_ref[...]) + bo_ref[...]


def _attention(q, k, v, ck, cv, sk, sv, selpos, g, Wo, bo2):
    full = lambda shape: pl.BlockSpec(shape, lambda i: (0, 0))
    return pl.pallas_call(
        _attn_body,
        grid=(NI,),
        in_specs=[
            pl.BlockSpec((BQ, D), lambda i: (i, 0)),
            pl.BlockSpec((BQ, D), lambda i: (jnp.maximum(i - 1, 0), 0)),
            pl.BlockSpec((BQ, D), lambda i: (i, 0)),
            pl.BlockSpec((BQ, D), lambda i: (jnp.maximum(i - 1, 0), 0)),
            pl.BlockSpec((BQ, D), lambda i: (i, 0)),
            full((NBLK, D)),
            full((NBLK, D)),
            full((TK * SB, D)),
            full((TK * SB, D)),
            full((1, TK * SB)),
            pl.BlockSpec((BQ, 3), lambda i: (i, 0)),
            full((D, D)),
            full((1, D)),
        ],
        out_specs=pl.BlockSpec((BQ, D), lambda i: (i, 0)),
        out_shape=jax.ShapeDtypeStruct((S, D), jnp.float32),
    )(q, k, k, v, v, ck, cv, sk, sv, selpos, g, Wo, bo2)


def kernel(x, Wq, Wk, Wv, Wo, bo, wk_comp, wv_comp, w_pe, Wg, bg):
    x2 = x[0]
    q, k, v, ck, cv, g, imp = _projections(
        x2, Wq, Wk, Wv, Wg, bg[None, :], wk_comp, wv_comp, w_pe)
    selpos, sk, sv = _topk_gather(imp.reshape(NBLK), k, v)
    out = _attention(q, k, v, ck, cv, sk, sv,
                     selpos.reshape(1, TK * SB), g, Wo, bo[None, :])
    return out[None]
